# baseline (device time: 46112 ns/iter reference)
import jax
import jax.numpy as jnp
from jax import lax
from jax.experimental import pallas as pl
from jax.experimental.pallas import tpu as pltpu

B, S, H, D = 2, 512, 8, 64
BH = B * H
SCALE = D ** -0.5
CHUNK = 32
NCHUNK = S // CHUNK
HALF = NCHUNK // 2


def kernel(Q, K, V):
    Qs = jnp.transpose((Q * SCALE).astype(jnp.bfloat16), (0, 2, 1, 3)).reshape(BH, S, D)
    Kb = jnp.transpose(K.astype(jnp.bfloat16), (0, 2, 1, 3)).reshape(BH, S, D)
    Vb = jnp.transpose(V.astype(jnp.bfloat16), (0, 2, 1, 3)).reshape(BH, S, D)
    Kc = jnp.transpose(Kb.reshape(BH, NCHUNK, CHUNK, D), (1, 0, 2, 3))
    Vc = jnp.transpose(Vb.reshape(BH, NCHUNK, CHUNK, D), (1, 0, 2, 3))
    KVc = jnp.stack([Kc, Vc], axis=1)

    def body(q_ref, k_ref, v_ref, kv_ref, out_ref,
             kv_recv, o_acc, ysend_sems, xsend_sems, recv_sems):
        my_x = lax.axis_index("x")
        my_y = lax.axis_index("y")
        my_z = lax.axis_index("z")
        y_partner = (my_x, 1 - my_y, my_z)
        x_partner = (1 - my_x, my_y, my_z)

        barrier_sem = pltpu.get_barrier_semaphore()
        for nbr in (y_partner, x_partner):
            pl.semaphore_signal(
                barrier_sem, inc=1,
                device_id=nbr, device_id_type=pl.DeviceIdType.MESH,
            )
        pl.semaphore_wait(barrier_sem, 2)

        base = my_x * HALF
        obase = (1 - my_x) * HALF

        y_rdmas = []
        for jj in range(HALF):
            r = pltpu.make_async_remote_copy(
                src_ref=kv_ref.at[base + jj],
                dst_ref=kv_recv.at[base + jj],
                send_sem=ysend_sems.at[jj],
                recv_sem=recv_sems.at[base + jj],
                device_id=y_partner,
                device_id_type=pl.DeviceIdType.MESH,
            )
            r.start()
            y_rdmas.append(r)

        ones = jnp.ones((S, 1), jnp.bfloat16)
        for i in range(BH):
            s1 = lax.dot_general(
                q_ref[i], k_ref[i], (((1,), (1,)), ((), ())),
                preferred_element_type=jnp.float32,
            )
            p1 = jnp.exp(s1).astype(jnp.bfloat16)
            vaug = jnp.concatenate([v_ref[i], ones], axis=1)
            o_acc[i] = lax.dot_general(
                p1, vaug, (((1,), (0,)), ((), ())),
                preferred_element_type=jnp.float32,
            )

        x_rdmas = []
        for jj in range(HALF):
            y_rdmas[jj].wait_recv()
            r = pltpu.make_async_remote_copy(
                src_ref=kv_recv.at[base + jj],
                dst_ref=kv_recv.at[base + jj],
                send_sem=xsend_sems.at[jj],
                recv_sem=recv_sems.at[base + jj],
                device_id=x_partner,
                device_id_type=pl.DeviceIdType.MESH,
            )
            r.start()
            x_rdmas.append(r)

        onesh = jnp.ones((S // 2, 1), jnp.bfloat16)
        for i in range(BH):
            kh = jnp.concatenate(
                [kv_recv[base + jj, 0, i] for jj in range(HALF)], axis=0)
            vh = jnp.concatenate(
                [kv_recv[base + jj, 1, i] for jj in range(HALF)], axis=0)
            s2 = lax.dot_general(
                q_ref[i], kh, (((1,), (1,)), ((), ())),
                preferred_element_type=jnp.float32,
            )
            p2 = jnp.exp(s2).astype(jnp.bfloat16)
            vaug = jnp.concatenate([vh, onesh], axis=1)
            o_acc[i] = o_acc[i] + lax.dot_general(
                p2, vaug, (((1,), (0,)), ((), ())),
                preferred_element_type=jnp.float32,
            )

        for jj in range(HALF):
            r = pltpu.make_async_remote_copy(
                src_ref=kv_recv.at[obase + jj],
                dst_ref=kv_recv.at[obase + jj],
                send_sem=xsend_sems.at[jj],
                recv_sem=recv_sems.at[obase + jj],
                device_id=x_partner,
                device_id_type=pl.DeviceIdType.MESH,
            )
            r.wait_recv()

        for i in range(BH):
            kh = jnp.concatenate(
                [kv_recv[obase + jj, 0, i] for jj in range(HALF)], axis=0)
            vh = jnp.concatenate(
                [kv_recv[obase + jj, 1, i] for jj in range(HALF)], axis=0)
            s3 = lax.dot_general(
                q_ref[i], kh, (((1,), (1,)), ((), ())),
                preferred_element_type=jnp.float32,
            )
            p3 = jnp.exp(s3).astype(jnp.bfloat16)
            vaug = jnp.concatenate([vh, onesh], axis=1)
            acc = o_acc[i] + lax.dot_general(
                p3, vaug, (((1,), (0,)), ((), ())),
                preferred_element_type=jnp.float32,
            )
            out_ref[i] = acc[:, :D] / acc[:, D:D + 1]

        for jj in range(HALF):
            y_rdmas[jj].wait_send()
            x_rdmas[jj].wait_send()

    out = pl.pallas_call(
        body,
        out_shape=jax.ShapeDtypeStruct((BH, S, D), jnp.float32),
        in_specs=[pl.BlockSpec(memory_space=pltpu.VMEM)] * 4,
        out_specs=pl.BlockSpec(memory_space=pltpu.VMEM),
        scratch_shapes=[
            pltpu.VMEM((NCHUNK, 2, BH, CHUNK, D), jnp.bfloat16),
            pltpu.VMEM((BH, S, D + 1), jnp.float32),
            pltpu.SemaphoreType.DMA((HALF,)),
            pltpu.SemaphoreType.DMA((HALF,)),
            pltpu.SemaphoreType.DMA((NCHUNK,)),
        ],
        compiler_params=pltpu.CompilerParams(collective_id=0),
    )(Qs, Kb, Vb, KVc)

    return jnp.transpose(out.reshape(B, H, S, D), (0, 2, 1, 3))


# device time: 44216 ns/iter; 1.0429x vs baseline; 1.0429x over previous
import jax
import jax.numpy as jnp
from jax import lax
from jax.experimental import pallas as pl
from jax.experimental.pallas import tpu as pltpu

B, S, H, D = 2, 512, 8, 64
BH = B * H
SCALE = D ** -0.5
CHUNK = 32
NCHUNK = S // CHUNK
HALF = NCHUNK // 2
GRP = 4
GROWS = GRP * CHUNK


def kernel(Q, K, V):
    Qs = jnp.transpose((Q * SCALE).astype(jnp.bfloat16), (0, 2, 1, 3)).reshape(BH, S, D)
    Kb = jnp.transpose(K.astype(jnp.bfloat16), (0, 2, 1, 3)).reshape(BH, S, D)
    Vb = jnp.transpose(V.astype(jnp.bfloat16), (0, 2, 1, 3)).reshape(BH, S, D)
    Kc = jnp.transpose(Kb.reshape(BH, NCHUNK, CHUNK, D), (1, 0, 2, 3))
    Vc = jnp.transpose(Vb.reshape(BH, NCHUNK, CHUNK, D), (1, 0, 2, 3))
    KVc = jnp.stack([Kc, Vc], axis=1)

    def body(q_ref, k_ref, v_ref, kv_ref, out_ref,
             kv_recv, o_acc, ysend_sems, xsend_sems, recv_sems):
        my_x = lax.axis_index("x")
        my_y = lax.axis_index("y")
        my_z = lax.axis_index("z")
        y_partner = (my_x, 1 - my_y, my_z)
        x_partner = (1 - my_x, my_y, my_z)

        barrier_sem = pltpu.get_barrier_semaphore()
        for nbr in (y_partner, x_partner):
            pl.semaphore_signal(
                barrier_sem, inc=1,
                device_id=nbr, device_id_type=pl.DeviceIdType.MESH,
            )
        pl.semaphore_wait(barrier_sem, 2)

        base = my_x * HALF
        obase = (1 - my_x) * HALF

        y_rdmas = []
        for jj in range(HALF):
            r = pltpu.make_async_remote_copy(
                src_ref=kv_ref.at[base + jj],
                dst_ref=kv_recv.at[base + jj],
                send_sem=ysend_sems.at[jj],
                recv_sem=recv_sems.at[base + jj],
                device_id=y_partner,
                device_id_type=pl.DeviceIdType.MESH,
            )
            r.start()
            y_rdmas.append(r)

        ones = jnp.ones((S, 1), jnp.bfloat16)
        onesg = jnp.ones((GROWS, 1), jnp.bfloat16)

        def local_head(i):
            s1 = lax.dot_general(
                q_ref[i], k_ref[i], (((1,), (1,)), ((), ())),
                preferred_element_type=jnp.float32,
            )
            p1 = jnp.exp(s1).astype(jnp.bfloat16)
            vaug = jnp.concatenate([v_ref[i], ones], axis=1)
            o_acc[i] = lax.dot_general(
                p1, vaug, (((1,), (0,)), ((), ())),
                preferred_element_type=jnp.float32,
            )

        def attend_group(first_chunk, final):
            for i in range(BH):
                kh = jnp.concatenate(
                    [kv_recv[first_chunk + c, 0, i] for c in range(GRP)], axis=0)
                vh = jnp.concatenate(
                    [kv_recv[first_chunk + c, 1, i] for c in range(GRP)], axis=0)
                s = lax.dot_general(
                    q_ref[i], kh, (((1,), (1,)), ((), ())),
                    preferred_element_type=jnp.float32,
                )
                p = jnp.exp(s).astype(jnp.bfloat16)
                vaug = jnp.concatenate([vh, onesg], axis=1)
                o = lax.dot_general(
                    p, vaug, (((1,), (0,)), ((), ())),
                    preferred_element_type=jnp.float32,
                )
                if final:
                    acc = o_acc[i] + o
                    out_ref[i] = acc[:, :D] / acc[:, D:D + 1]
                else:
                    o_acc[i] = o_acc[i] + o

        def wait_x_chunk(jj):
            r = pltpu.make_async_remote_copy(
                src_ref=kv_recv.at[obase + jj],
                dst_ref=kv_recv.at[obase + jj],
                send_sem=xsend_sems.at[jj],
                recv_sem=recv_sems.at[obase + jj],
                device_id=x_partner,
                device_id_type=pl.DeviceIdType.MESH,
            )
            r.wait_recv()

        x_rdmas = []
        for jj in range(HALF):
            y_rdmas[jj].wait_recv()
            r = pltpu.make_async_remote_copy(
                src_ref=kv_recv.at[base + jj],
                dst_ref=kv_recv.at[base + jj],
                send_sem=xsend_sems.at[jj],
                recv_sem=recv_sems.at[base + jj],
                device_id=x_partner,
                device_id_type=pl.DeviceIdType.MESH,
            )
            r.start()
            x_rdmas.append(r)
            local_head(2 * jj)
            local_head(2 * jj + 1)
            if jj == GRP - 1:
                attend_group(base, final=False)
            if jj == GRP + 1:
                for c in range(GRP):
                    wait_x_chunk(c)
                attend_group(obase, final=False)

        attend_group(base + GRP, final=False)
        for c in range(GRP, 2 * GRP):
            wait_x_chunk(c)
        attend_group(obase + GRP, final=True)

        for jj in range(HALF):
            y_rdmas[jj].wait_send()
            x_rdmas[jj].wait_send()

    out = pl.pallas_call(
        body,
        out_shape=jax.ShapeDtypeStruct((BH, S, D), jnp.float32),
        in_specs=[pl.BlockSpec(memory_space=pltpu.VMEM)] * 4,
        out_specs=pl.BlockSpec(memory_space=pltpu.VMEM),
        scratch_shapes=[
            pltpu.VMEM((NCHUNK, 2, BH, CHUNK, D), jnp.bfloat16),
            pltpu.VMEM((BH, S, D + 1), jnp.float32),
            pltpu.SemaphoreType.DMA((HALF,)),
            pltpu.SemaphoreType.DMA((HALF,)),
            pltpu.SemaphoreType.DMA((NCHUNK,)),
        ],
        compiler_params=pltpu.CompilerParams(collective_id=0),
    )(Qs, Kb, Vb, KVc)

    return jnp.transpose(out.reshape(B, H, S, D), (0, 2, 1, 3))
